# baseline (speedup 1.0000x reference)
import dataclasses
import functools

import jax
import jax.numpy as jnp
from jax import lax
from jax.experimental import pallas as pl
from jax.experimental.pallas import tpu as pltpu
from jax.experimental.pallas import tpu_sc as plsc

_ROWS = 16384
_COLS = 200
_N = _ROWS * _COLS
_NUM_WORKERS = 32
_PER_W = _N // _NUM_WORKERS

_mesh = plsc.VectorSubcoreMesh(core_axis_name="c", subcore_axis_name="s")

_cp = pltpu.CompilerParams()
if "needs_layout_passes" in pltpu.CompilerParams.__dataclass_fields__:
    _cp = dataclasses.replace(_cp, needs_layout_passes=False)


@functools.partial(
    pl.kernel,
    out_type=jax.ShapeDtypeStruct((_N,), jnp.float32),
    mesh=_mesh,
    compiler_params=_cp,
    scratch_types=[
        pltpu.VMEM((_PER_W,), jnp.float32),
        pltpu.VMEM((24,), jnp.float32),
    ],
)
def _step_lookup(x_hbm, tab_hbm, out_hbm, buf, tab_v):
    wid = lax.axis_index("c") * 16 + lax.axis_index("s")
    base = wid * _PER_W
    pltpu.sync_copy(tab_hbm, tab_v.at[pl.ds(0, 17)])
    pltpu.sync_copy(x_hbm.at[pl.ds(base, _PER_W)], buf)

    @pl.loop(0, _PER_W, step=16)
    def _(i):
        v = buf[pl.ds(i, 16)]
        vc = jnp.minimum(jnp.maximum(v, -8.0), 8.0)
        idx = (vc + 8.0).astype(jnp.int32)
        buf[pl.ds(i, 16)] = plsc.load_gather(tab_v, [idx])

    pltpu.sync_copy(buf, out_hbm.at[pl.ds(base, _PER_W)])


@jax.jit
def kernel(x, function_values):
    out = _step_lookup(x.reshape(_N), function_values)
    return out.reshape(x.shape)


# speedup vs baseline: 1.7974x; 1.7974x over previous
import dataclasses
import functools

import jax
import jax.numpy as jnp
from jax import lax
from jax.experimental import pallas as pl
from jax.experimental.pallas import tpu as pltpu
from jax.experimental.pallas import tpu_sc as plsc

_ROWS = 16384
_COLS = 200
_N = _ROWS * _COLS
_NUM_WORKERS = 32
_PER_W = _N // _NUM_WORKERS
_C = 12800
_NCH = _PER_W // _C

_mesh = plsc.VectorSubcoreMesh(core_axis_name="c", subcore_axis_name="s")

_cp = pltpu.CompilerParams()
if "needs_layout_passes" in pltpu.CompilerParams.__dataclass_fields__:
    _cp = dataclasses.replace(_cp, needs_layout_passes=False)


@functools.partial(
    pl.kernel,
    out_type=jax.ShapeDtypeStruct((_N,), jnp.float32),
    mesh=_mesh,
    compiler_params=_cp,
    scratch_types=[
        pltpu.VMEM((_C,), jnp.float32),
        pltpu.VMEM((_C,), jnp.float32),
        pltpu.VMEM((_C,), jnp.float32),
        pltpu.VMEM((_C,), jnp.float32),
        pltpu.VMEM((24,), jnp.float32),
        pltpu.SemaphoreType.DMA,
        pltpu.SemaphoreType.DMA,
        pltpu.SemaphoreType.DMA,
        pltpu.SemaphoreType.DMA,
    ],
)
def _step_lookup(x_hbm, tab_hbm, out_hbm, inb0, inb1, outb0, outb1, tab_v,
                 in_sem0, in_sem1, out_sem0, out_sem1):
    wid = lax.axis_index("c") * 16 + lax.axis_index("s")
    base = wid * _PER_W
    in_bufs = (inb0, inb1)
    out_bufs = (outb0, outb1)
    in_sems = (in_sem0, in_sem1)
    out_sems = (out_sem0, out_sem1)

    pltpu.sync_copy(tab_hbm, tab_v.at[pl.ds(0, 17)])

    def in_copy(g):
        b = g % 2
        return pltpu.make_async_copy(
            x_hbm.at[pl.ds(base + g * _C, _C)], in_bufs[b], in_sems[b])

    def out_copy(g):
        b = g % 2
        return pltpu.make_async_copy(
            out_bufs[b], out_hbm.at[pl.ds(base + g * _C, _C)], out_sems[b])

    in_copy(0).start()
    in_copy(1).start()

    for g in range(_NCH):
        b = g % 2
        in_copy(g).wait()
        if g >= 2:
            out_copy(g - 2).wait()
        ib = in_bufs[b]
        ob = out_bufs[b]

        @plsc.parallel_loop(0, _C, step=16, unroll=8)
        def _(i, _ib=ib, _ob=ob):
            v = _ib[pl.ds(i, 16)]
            vc = jnp.minimum(jnp.maximum(v, -8.0), 8.0)
            idx = (vc + 8.0).astype(jnp.int32)
            _ob[pl.ds(i, 16)] = plsc.load_gather(tab_v, [idx])

        out_copy(g).start()
        if g + 2 < _NCH:
            in_copy(g + 2).start()

    out_copy(_NCH - 2).wait()
    out_copy(_NCH - 1).wait()


@jax.jit
def kernel(x, function_values):
    out = _step_lookup(x.reshape(_N), function_values)
    return out.reshape(x.shape)


# speedup vs baseline: 3.1046x; 1.7273x over previous
import dataclasses
import functools

import jax
import jax.numpy as jnp
from jax import lax
from jax.experimental import pallas as pl
from jax.experimental.pallas import tpu as pltpu
from jax.experimental.pallas import tpu_sc as plsc

_ROWS = 16384
_COLS = 200
_NUM_WORKERS = 32
_ROWS_W = _ROWS // _NUM_WORKERS
_CR = 64
_NCH = _ROWS_W // _CR
_CSTARTS = tuple(range(0, _COLS - 16, 16)) + (_COLS - 16,)

_mesh = plsc.VectorSubcoreMesh(core_axis_name="c", subcore_axis_name="s")

_cp = pltpu.CompilerParams()
if "needs_layout_passes" in pltpu.CompilerParams.__dataclass_fields__:
    _cp = dataclasses.replace(_cp, needs_layout_passes=False)


@functools.partial(
    pl.kernel,
    out_type=jax.ShapeDtypeStruct((_ROWS, _COLS), jnp.float32),
    mesh=_mesh,
    compiler_params=_cp,
    scratch_types=[
        pltpu.VMEM((_CR, _COLS), jnp.float32),
        pltpu.VMEM((_CR, _COLS), jnp.float32),
        pltpu.VMEM((_CR, _COLS), jnp.float32),
        pltpu.VMEM((_CR, _COLS), jnp.float32),
        pltpu.VMEM((24,), jnp.float32),
        pltpu.SemaphoreType.DMA,
        pltpu.SemaphoreType.DMA,
        pltpu.SemaphoreType.DMA,
        pltpu.SemaphoreType.DMA,
    ],
)
def _step_lookup(x_hbm, tab_hbm, out_hbm, inb0, inb1, outb0, outb1, tab_v,
                 in_sem0, in_sem1, out_sem0, out_sem1):
    wid = lax.axis_index("c") * 16 + lax.axis_index("s")
    base = wid * _ROWS_W
    in_bufs = (inb0, inb1)
    out_bufs = (outb0, outb1)
    in_sems = (in_sem0, in_sem1)
    out_sems = (out_sem0, out_sem1)

    pltpu.sync_copy(tab_hbm, tab_v.at[pl.ds(0, 17)])

    def in_copy(g):
        b = g % 2
        return pltpu.make_async_copy(
            x_hbm.at[pl.ds(base + g * _CR, _CR), :], in_bufs[b], in_sems[b])

    def out_copy(g):
        b = g % 2
        return pltpu.make_async_copy(
            out_bufs[b], out_hbm.at[pl.ds(base + g * _CR, _CR), :],
            out_sems[b])

    in_copy(0).start()
    in_copy(1).start()

    for g in range(_NCH):
        b = g % 2
        in_copy(g).wait()
        if g >= 2:
            out_copy(g - 2).wait()
        ib = in_bufs[b]
        ob = out_bufs[b]

        @plsc.parallel_loop(0, _CR, step=1)
        def _(r, _ib=ib, _ob=ob):
            for c in _CSTARTS:
                v = _ib[r, pl.ds(c, 16)]
                vc = jnp.minimum(jnp.maximum(v, -8.0), 8.0)
                idx = (vc + 8.0).astype(jnp.int32)
                _ob[r, pl.ds(c, 16)] = plsc.load_gather(tab_v, [idx])

        out_copy(g).start()
        if g + 2 < _NCH:
            in_copy(g + 2).start()

    out_copy(_NCH - 2).wait()
    out_copy(_NCH - 1).wait()


@jax.jit
def kernel(x, function_values):
    return _step_lookup(x, function_values)


# speedup vs baseline: 3.1122x; 1.0024x over previous
import dataclasses
import functools

import jax
import jax.numpy as jnp
from jax import lax
from jax.experimental import pallas as pl
from jax.experimental.pallas import tpu as pltpu
from jax.experimental.pallas import tpu_sc as plsc

_ROWS = 16384
_COLS = 200
_NUM_WORKERS = 32
_ROWS_W = _ROWS // _NUM_WORKERS
_CR = 64
_NCH = _ROWS_W // _CR
_CSTARTS = tuple(range(0, _COLS - 16, 16)) + (_COLS - 16,)

_mesh = plsc.VectorSubcoreMesh(core_axis_name="c", subcore_axis_name="s")

_cp = pltpu.CompilerParams()
if "needs_layout_passes" in pltpu.CompilerParams.__dataclass_fields__:
    _cp = dataclasses.replace(_cp, needs_layout_passes=False)
_cp = dataclasses.replace(_cp, use_tc_tiling_on_sc=True)


@functools.partial(
    pl.kernel,
    out_type=jax.ShapeDtypeStruct((_ROWS, _COLS), jnp.float32),
    mesh=_mesh,
    compiler_params=_cp,
    scratch_types=[
        pltpu.VMEM((_CR, _COLS), jnp.float32),
        pltpu.VMEM((_CR, _COLS), jnp.float32),
        pltpu.VMEM((_CR, _COLS), jnp.float32),
        pltpu.VMEM((_CR, _COLS), jnp.float32),
        pltpu.VMEM((24,), jnp.float32),
        pltpu.SemaphoreType.DMA,
        pltpu.SemaphoreType.DMA,
        pltpu.SemaphoreType.DMA,
        pltpu.SemaphoreType.DMA,
    ],
)
def _step_lookup(x_hbm, tab_hbm, out_hbm, inb0, inb1, outb0, outb1, tab_v,
                 in_sem0, in_sem1, out_sem0, out_sem1):
    wid = lax.axis_index("c") * 16 + lax.axis_index("s")
    base = wid * _ROWS_W
    in_bufs = (inb0, inb1)
    out_bufs = (outb0, outb1)
    in_sems = (in_sem0, in_sem1)
    out_sems = (out_sem0, out_sem1)

    pltpu.sync_copy(tab_hbm, tab_v.at[pl.ds(0, 17)])

    def in_copy(g):
        b = g % 2
        return pltpu.make_async_copy(
            x_hbm.at[pl.ds(base + g * _CR, _CR), :], in_bufs[b], in_sems[b])

    def out_copy(g):
        b = g % 2
        return pltpu.make_async_copy(
            out_bufs[b], out_hbm.at[pl.ds(base + g * _CR, _CR), :],
            out_sems[b])

    in_copy(0).start()
    in_copy(1).start()

    for g in range(_NCH):
        b = g % 2
        in_copy(g).wait()
        if g >= 2:
            out_copy(g - 2).wait()
        ib = in_bufs[b]
        ob = out_bufs[b]

        @plsc.parallel_loop(0, _CR, step=1)
        def _(r, _ib=ib, _ob=ob):
            for c in _CSTARTS:
                v = _ib[r, pl.ds(c, 16)]
                vc = jnp.minimum(jnp.maximum(v, -8.0), 8.0)
                idx = (vc + 8.0).astype(jnp.int32)
                _ob[r, pl.ds(c, 16)] = plsc.load_gather(tab_v, [idx])

        out_copy(g).start()
        if g + 2 < _NCH:
            in_copy(g + 2).start()

    out_copy(_NCH - 2).wait()
    out_copy(_NCH - 1).wait()


@jax.jit
def kernel(x, function_values):
    return _step_lookup(x, function_values)


# speedup vs baseline: 5.5762x; 1.7917x over previous
import dataclasses
import functools

import jax
import jax.numpy as jnp
from jax import lax
from jax.experimental import pallas as pl
from jax.experimental.pallas import tpu as pltpu
from jax.experimental.pallas import tpu_sc as plsc

_ROWS = 200
_COLS = 16384
_NUM_WORKERS = 32
_COLS_W = _COLS // _NUM_WORKERS
_CC = 128
_NCH = _COLS_W // _CC

_mesh = plsc.VectorSubcoreMesh(core_axis_name="c", subcore_axis_name="s")

_cp = pltpu.CompilerParams()
if "needs_layout_passes" in pltpu.CompilerParams.__dataclass_fields__:
    _cp = dataclasses.replace(_cp, needs_layout_passes=False)
_cp = dataclasses.replace(_cp, use_tc_tiling_on_sc=True)


@functools.partial(
    pl.kernel,
    out_type=jax.ShapeDtypeStruct((_ROWS, _COLS), jnp.float32),
    mesh=_mesh,
    compiler_params=_cp,
    scratch_types=[
        pltpu.VMEM((_ROWS, _CC), jnp.float32),
        pltpu.VMEM((_ROWS, _CC), jnp.float32),
        pltpu.VMEM((_ROWS, _CC), jnp.float32),
        pltpu.VMEM((_ROWS, _CC), jnp.float32),
        pltpu.VMEM((24,), jnp.float32),
        pltpu.SemaphoreType.DMA,
        pltpu.SemaphoreType.DMA,
        pltpu.SemaphoreType.DMA,
        pltpu.SemaphoreType.DMA,
    ],
)
def _step_lookup(x_hbm, tab_hbm, out_hbm, inb0, inb1, outb0, outb1, tab_v,
                 in_sem0, in_sem1, out_sem0, out_sem1):
    wid = lax.axis_index("c") * 16 + lax.axis_index("s")
    base = wid * _COLS_W
    in_bufs = (inb0, inb1)
    out_bufs = (outb0, outb1)
    in_sems = (in_sem0, in_sem1)
    out_sems = (out_sem0, out_sem1)

    pltpu.sync_copy(tab_hbm, tab_v.at[pl.ds(0, 17)])

    def in_copy(g):
        b = g % 2
        return pltpu.make_async_copy(
            x_hbm.at[:, pl.ds(base + g * _CC, _CC)], in_bufs[b], in_sems[b])

    def out_copy(g):
        b = g % 2
        return pltpu.make_async_copy(
            out_bufs[b], out_hbm.at[:, pl.ds(base + g * _CC, _CC)],
            out_sems[b])

    in_copy(0).start()
    in_copy(1).start()

    for g in range(_NCH):
        b = g % 2
        in_copy(g).wait()
        if g >= 2:
            out_copy(g - 2).wait()
        ib = in_bufs[b]
        ob = out_bufs[b]

        @plsc.parallel_loop(0, _ROWS, step=1)
        def _(r, _ib=ib, _ob=ob):
            for c in range(0, _CC, 16):
                v = _ib[r, pl.ds(c, 16)]
                vc = jnp.minimum(jnp.maximum(v, -8.0), 8.0)
                idx = (vc + 8.0).astype(jnp.int32)
                _ob[r, pl.ds(c, 16)] = plsc.load_gather(tab_v, [idx])

        out_copy(g).start()
        if g + 2 < _NCH:
            in_copy(g + 2).start()

    out_copy(_NCH - 2).wait()
    out_copy(_NCH - 1).wait()


@jax.jit
def kernel(x, function_values):
    return _step_lookup(x.T, function_values).T


# speedup vs baseline: 5.8277x; 1.0451x over previous
import dataclasses
import functools

import jax
import jax.numpy as jnp
from jax import lax
from jax.experimental import pallas as pl
from jax.experimental.pallas import tpu as pltpu
from jax.experimental.pallas import tpu_sc as plsc

_ROWS = 200
_COLS = 16384
_NUM_WORKERS = 32
_COLS_W = _COLS // _NUM_WORKERS
_CC = 128
_NCH = _COLS_W // _CC

_mesh = plsc.VectorSubcoreMesh(core_axis_name="c", subcore_axis_name="s")

_cp = pltpu.CompilerParams()
if "needs_layout_passes" in pltpu.CompilerParams.__dataclass_fields__:
    _cp = dataclasses.replace(_cp, needs_layout_passes=False)
_cp = dataclasses.replace(_cp, use_tc_tiling_on_sc=True)


@functools.partial(
    pl.kernel,
    out_type=jax.ShapeDtypeStruct((_ROWS, _COLS), jnp.float32),
    mesh=_mesh,
    compiler_params=_cp,
    scratch_types=[
        pltpu.VMEM((_ROWS, _CC), jnp.float32),
        pltpu.VMEM((_ROWS, _CC), jnp.float32),
        pltpu.VMEM((_ROWS, _CC), jnp.float32),
        pltpu.VMEM((_ROWS, _CC), jnp.float32),
        pltpu.VMEM((24,), jnp.float32),
        pltpu.SemaphoreType.DMA,
        pltpu.SemaphoreType.DMA,
        pltpu.SemaphoreType.DMA,
        pltpu.SemaphoreType.DMA,
    ],
)
def _step_lookup(x_hbm, tab_hbm, out_hbm, inb0, inb1, outb0, outb1, tab_v,
                 in_sem0, in_sem1, out_sem0, out_sem1):
    wid = lax.axis_index("c") * 16 + lax.axis_index("s")
    base = wid * _COLS_W
    in_bufs = (inb0, inb1)
    out_bufs = (outb0, outb1)
    in_sems = (in_sem0, in_sem1)
    out_sems = (out_sem0, out_sem1)

    def in_copy(g, b):
        return pltpu.make_async_copy(
            x_hbm.at[:, pl.ds(base + g * _CC, _CC)], in_bufs[b], in_sems[b])

    def out_copy(g, b):
        return pltpu.make_async_copy(
            out_bufs[b], out_hbm.at[:, pl.ds(base + g * _CC, _CC)],
            out_sems[b])

    in_copy(0, 0).start()
    in_copy(1, 1).start()
    pltpu.sync_copy(tab_hbm, tab_v.at[pl.ds(0, 17)])

    @pl.loop(0, _NCH, step=2)
    def _(g0):
        for p in range(2):
            g = g0 + p
            in_copy(g, p).wait()

            @pl.when(g >= 2)
            def _():
                out_copy(g - 2, p).wait()

            ib = in_bufs[p]
            ob = out_bufs[p]

            @plsc.parallel_loop(0, _ROWS, step=1)
            def _(r, _ib=ib, _ob=ob):
                for c in range(0, _CC, 16):
                    v = _ib[r, pl.ds(c, 16)]
                    vc = jnp.minimum(jnp.maximum(v, -8.0), 8.0)
                    idx = (vc + 8.0).astype(jnp.int32)
                    _ob[r, pl.ds(c, 16)] = plsc.load_gather(tab_v, [idx])

            out_copy(g, p).start()

            @pl.when(g + 2 < _NCH)
            def _():
                in_copy(g + 2, p).start()

    out_copy(_NCH - 2, 0).wait()
    out_copy(_NCH - 1, 1).wait()


@jax.jit
def kernel(x, function_values):
    return _step_lookup(x.T, function_values).T
